# baseline (speedup 1.0000x reference)
import jax
import jax.numpy as jnp
from jax.experimental import pallas as pl
from jax.experimental.pallas import tpu as pltpu


def _concat_kernel(hs_ref, emb_ref, out_ref,
                   cls_vmem, tail_vmem, cls_sem, copy_sem, tail_sem):
    q = pl.program_id(0)
    depth = cls_vmem.shape[0]
    seq_len = hs_ref.shape[1]

    cls_copy = pltpu.make_async_copy(
        hs_ref.at[pl.ds(q * depth, depth), pl.ds(0, 8), :], cls_vmem, cls_sem)
    cls_copy.start()

    doc_copies = []
    for d in range(depth):
        c = pltpu.make_async_copy(
            hs_ref.at[q * depth + d],
            out_ref.at[q * depth + d, pl.ds(0, seq_len), :],
            copy_sem)
        c.start()
        doc_copies.append(c)

    cls_copy.wait()
    tail_vmem[...] = cls_vmem[:, 1, :] + emb_ref[0]

    tail_copies = []
    for d in range(depth):
        c = pltpu.make_async_copy(
            tail_vmem,
            out_ref.at[q * depth + d, pl.ds(seq_len, depth), :],
            tail_sem)
        c.start()
        tail_copies.append(c)

    for c in doc_copies:
        c.wait()
    for c in tail_copies:
        c.wait()


def kernel(hidden_states, other_seq_emb, num_docs):
    total_docs, seq_len, hidden = hidden_states.shape
    n_queries = num_docs.shape[0]
    depth = total_docs // n_queries
    out = pl.pallas_call(
        _concat_kernel,
        grid=(n_queries,),
        in_specs=[
            pl.BlockSpec(memory_space=pl.ANY),
            pl.BlockSpec((1, hidden), lambda q: (0, 0)),
        ],
        out_specs=pl.BlockSpec(memory_space=pl.ANY),
        out_shape=jax.ShapeDtypeStruct(
            (total_docs, seq_len + depth, hidden), hidden_states.dtype),
        scratch_shapes=[
            pltpu.VMEM((depth, 8, hidden), hidden_states.dtype),
            pltpu.VMEM((depth, hidden), hidden_states.dtype),
            pltpu.SemaphoreType.DMA,
            pltpu.SemaphoreType.DMA,
            pltpu.SemaphoreType.DMA,
        ],
    )(hidden_states, other_seq_emb)
    return out


# speedup vs baseline: 47.1455x; 47.1455x over previous
import jax
import jax.numpy as jnp
from jax.experimental import pallas as pl

_BD = 8


def _concat_kernel(hs_ref, cls_ref, emb_ref, out_ref):
    seq_len = hs_ref.shape[1]
    out_ref[:, :seq_len, :] = hs_ref[...]
    tail = cls_ref[:, 1, :] + emb_ref[0]
    out_ref[:, seq_len:, :] = jnp.broadcast_to(
        tail[None], (out_ref.shape[0],) + tail.shape
    )


def kernel(hidden_states, other_seq_emb, num_docs):
    total_docs, seq_len, hidden = hidden_states.shape
    n_queries = num_docs.shape[0]
    depth = total_docs // n_queries
    bd = _BD
    grid = (total_docs // bd,)
    blocks_per_query = depth // bd
    out = pl.pallas_call(
        _concat_kernel,
        grid=grid,
        in_specs=[
            pl.BlockSpec((bd, seq_len, hidden), lambda i: (i, 0, 0)),
            pl.BlockSpec((depth, 8, hidden),
                         lambda i: (i // blocks_per_query, 0, 0)),
            pl.BlockSpec((1, hidden), lambda i: (0, 0)),
        ],
        out_specs=pl.BlockSpec((bd, seq_len + depth, hidden),
                               lambda i: (i, 0, 0)),
        out_shape=jax.ShapeDtypeStruct(
            (total_docs, seq_len + depth, hidden), hidden_states.dtype),
    )(hidden_states, hidden_states, other_seq_emb)
    return out
